# baseline (speedup 1.0000x reference)
import functools

import jax
import jax.numpy as jnp
from jax import lax
from jax.experimental import pallas as pl
from jax.experimental.pallas import tpu as pltpu
from jax.experimental.pallas import tpu_sc as plsc

B = 8192
D_IN = 2048
H1 = 2048
H2 = 1024
LATENT = 256
K = 8192
BLK = 256
NBLK = B // BLK
BN_EPS = 1e-5
CC = 0.25

_PREC = jax.lax.Precision.HIGHEST

_NC = 2
_NS = 16
_NW = _NC * _NS
_BPW = B // _NW
_CH = 128
_CW = 16


def _leaky(h):
    return jnp.where(h >= 0, h, 0.2 * h)


def _enc_body(x_ref, w1_ref, s1_ref, t1_ref, w2_ref, s2_ref, t2_ref,
              w3_ref, b3_ref, z_ref):
    h = jnp.dot(x_ref[...], w1_ref[...], preferred_element_type=jnp.float32,
                precision=_PREC)
    h = _leaky(h * s1_ref[...] + t1_ref[...])
    h = jnp.dot(h, w2_ref[...], preferred_element_type=jnp.float32,
                precision=_PREC)
    h = _leaky(h * s2_ref[...] + t2_ref[...])
    z = jnp.dot(h, w3_ref[...], preferred_element_type=jnp.float32,
                precision=_PREC) + b3_ref[...]
    z_ref[...] = z


def _vq_body(z_ref, et_ref, idx_ref):
    et = et_ref[...]
    e_sq = jnp.sum(et * et, axis=0, keepdims=True)
    scores = e_sq - 2.0 * jnp.dot(z_ref[...], et,
                                  preferred_element_type=jnp.float32,
                                  precision=_PREC)
    m = jnp.min(scores, axis=1, keepdims=True)
    iota = lax.broadcasted_iota(jnp.int32, scores.shape, 1)
    idx = jnp.min(jnp.where(scores == m, iota, K), axis=1)
    idx_ref[...] = idx.reshape(1, 1, BLK)


def _dec_body(q_ref, z_ref, v1_ref, s1_ref, t1_ref, v2_ref, s2_ref, t2_ref,
              v3_ref, b3_ref, recon_ref, lp_ref):
    q = q_ref[...]
    d = q - z_ref[...]
    lp_ref[...] = jnp.broadcast_to(jnp.sum(d * d), (1, 1, 128))
    h = jnp.dot(q, v1_ref[...], preferred_element_type=jnp.float32,
                precision=_PREC)
    h = _leaky(h * s1_ref[...] + t1_ref[...])
    h = jnp.dot(h, v2_ref[...], preferred_element_type=jnp.float32,
                precision=_PREC)
    h = _leaky(h * s2_ref[...] + t2_ref[...])
    recon_ref[...] = jnp.dot(h, v3_ref[...], preferred_element_type=jnp.float32,
                             precision=_PREC) + b3_ref[...]


def _fin_body(lp_ref, c_ref, loss_ref, perp_ref):
    loss = jnp.sum(lp_ref[...]) * ((1.0 + CC) / (128.0 * B * LATENT))
    loss_ref[...] = loss.reshape(1, 1)
    c = c_ref[0] + c_ref[1]
    lane = lax.broadcasted_iota(jnp.int32, c.shape, 1)
    p = c * (1.0 / B)
    ent = jnp.where(lane == 0, p * jnp.log(p + 1e-10), 0.0)
    perp_ref[...] = jnp.exp(-jnp.sum(ent)).reshape(1, 1)


def _sc_body(table_hbm, idx_hbm, q_hbm, counts_hbm,
             idx_a, idx_b, rows_a, rows_b, ones_v, zeros_v, counts_sh,
             sem_a, sem_b):
    cid = lax.axis_index("c")
    sid = lax.axis_index("s")
    wid = sid * _NC + cid
    base = wid * _BPW

    pltpu.sync_copy(idx_hbm.at[pl.ds(base, _CH)], idx_a)
    pltpu.sync_copy(idx_hbm.at[pl.ds(base + _CH, _CH)], idx_b)
    cp_a = pltpu.async_copy(table_hbm.at[idx_a], rows_a, sem_a)
    cp_b = pltpu.async_copy(table_hbm.at[idx_b], rows_b, sem_b)

    @pl.loop(0, K // _NS)
    def _(r):
        zeros_v[r, :] = jnp.zeros((_CW,), jnp.float32)

    @pl.loop(0, _CH)
    def _(r):
        ones_v[r, :] = jnp.ones((_CW,), jnp.float32)

    pltpu.sync_copy(zeros_v, counts_sh.at[pl.ds(sid * (K // _NS), K // _NS)])
    plsc.subcore_barrier()

    pltpu.sync_copy(ones_v, counts_sh.at[idx_a], add=True)
    pltpu.sync_copy(ones_v, counts_sh.at[idx_b], add=True)
    plsc.subcore_barrier()

    pltpu.sync_copy(counts_sh.at[pl.ds(sid * (K // _NS), K // _NS)],
                    counts_hbm.at[cid, pl.ds(sid * (K // _NS), K // _NS)])

    cp_a.wait()
    cp_b.wait()
    pltpu.sync_copy(rows_a, q_hbm.at[pl.ds(base, _CH)])
    pltpu.sync_copy(rows_b, q_hbm.at[pl.ds(base + _CH, _CH)])


def _enc(x, w1, s1, t1, w2, s2, t2, w3, b3):
    full = lambda shape: pl.BlockSpec(shape, lambda i: tuple(0 for _ in shape))
    return pl.pallas_call(
        _enc_body,
        grid=(NBLK,),
        in_specs=[
            pl.BlockSpec((BLK, D_IN), lambda i: (i, 0)),
            full((D_IN, H1)), full((1, H1)), full((1, H1)),
            full((H1, H2)), full((1, H2)), full((1, H2)),
            full((H2, LATENT)), full((1, LATENT)),
        ],
        out_specs=pl.BlockSpec((BLK, LATENT), lambda i: (i, 0)),
        out_shape=jax.ShapeDtypeStruct((B, LATENT), jnp.float32),
    )(x, w1, s1, t1, w2, s2, t2, w3, b3)


def _vq(z, et):
    return pl.pallas_call(
        _vq_body,
        grid=(NBLK,),
        in_specs=[
            pl.BlockSpec((BLK, LATENT), lambda i: (i, 0)),
            pl.BlockSpec((LATENT, K), lambda i: (0, 0)),
        ],
        out_specs=pl.BlockSpec((1, 1, BLK), lambda i: (i, 0, 0)),
        out_shape=jax.ShapeDtypeStruct((NBLK, 1, BLK), jnp.int32),
    )(z, et)


def _sc_gather_counts(table, idx):
    k = pl.kernel(
        _sc_body,
        mesh=plsc.VectorSubcoreMesh(core_axis_name="c", subcore_axis_name="s"),
        out_type=(jax.ShapeDtypeStruct((B, LATENT), jnp.float32),
                  jax.ShapeDtypeStruct((_NC, K, _CW), jnp.float32)),
        scratch_types=[
            pltpu.VMEM((_CH,), jnp.int32),
            pltpu.VMEM((_CH,), jnp.int32),
            pltpu.VMEM((_CH, LATENT), jnp.float32),
            pltpu.VMEM((_CH, LATENT), jnp.float32),
            pltpu.VMEM((_CH, _CW), jnp.float32),
            pltpu.VMEM((K // _NS, _CW), jnp.float32),
            pltpu.VMEM_SHARED((K, _CW), jnp.float32),
            pltpu.SemaphoreType.DMA,
            pltpu.SemaphoreType.DMA,
        ],
        compiler_params=pltpu.CompilerParams(use_tc_tiling_on_sc=False),
    )
    return k(table, idx)


def _dec(q, z, v1, s1, t1, v2, s2, t2, v3, b3):
    full = lambda shape: pl.BlockSpec(shape, lambda i: tuple(0 for _ in shape))
    return pl.pallas_call(
        _dec_body,
        grid=(NBLK,),
        in_specs=[
            pl.BlockSpec((BLK, LATENT), lambda i: (i, 0)),
            pl.BlockSpec((BLK, LATENT), lambda i: (i, 0)),
            full((LATENT, H2)), full((1, H2)), full((1, H2)),
            full((H2, H1)), full((1, H1)), full((1, H1)),
            full((H1, D_IN)), full((1, D_IN)),
        ],
        out_specs=[
            pl.BlockSpec((BLK, D_IN), lambda i: (i, 0)),
            pl.BlockSpec((1, 1, 128), lambda i: (i, 0, 0)),
        ],
        out_shape=[
            jax.ShapeDtypeStruct((B, D_IN), jnp.float32),
            jax.ShapeDtypeStruct((NBLK, 1, 128), jnp.float32),
        ],
    )(q, z, v1, s1, t1, v2, s2, t2, v3, b3)


def _fin(lp, counts):
    return pl.pallas_call(
        _fin_body,
        out_shape=[
            jax.ShapeDtypeStruct((1, 1), jnp.float32),
            jax.ShapeDtypeStruct((1, 1), jnp.float32),
        ],
    )(lp, counts)


def _affine(layer):
    inv = 1.0 / jnp.sqrt(1.0 + BN_EPS)
    s = layer["gamma"] * inv
    t = layer["b"] * s + layer["beta"]
    return s[None, :], t[None, :]


def kernel(x, params):
    enc = params["enc"]
    dec = params["dec"]
    E = params["codebook"]

    s1, t1 = _affine(enc[0])
    s2, t2 = _affine(enc[1])
    z = _enc(x, enc[0]["W"].T, s1, t1, enc[1]["W"].T, s2, t2,
             enc[2]["W"].T, enc[2]["b"][None, :])

    idx3 = _vq(z, E.T)
    idx = idx3.reshape(B)

    q, counts = _sc_gather_counts(E, idx)

    d1, dt1 = _affine(dec[0])
    d2, dt2 = _affine(dec[1])
    recon, lp = _dec(q, z, dec[0]["W"].T, d1, dt1, dec[1]["W"].T, d2, dt2,
                     dec[2]["W"].T, dec[2]["b"][None, :])

    loss, perp = _fin(lp, counts)
    return recon, loss.reshape(()), perp.reshape(())


# speedup vs baseline: 3.8296x; 3.8296x over previous
import functools

import jax
import jax.numpy as jnp
from jax import lax
from jax.experimental import pallas as pl
from jax.experimental.pallas import tpu as pltpu
from jax.experimental.pallas import tpu_sc as plsc

B = 8192
D_IN = 2048
H1 = 2048
H2 = 1024
LATENT = 256
K = 8192
BLK = 256
NBLK = B // BLK
BN_EPS = 1e-5
CC = 0.25

_PREC = jax.lax.Precision.DEFAULT

_NC = 2
_NS = 16
_NW = _NC * _NS
_BPW = B // _NW
_CH = 128
_CW = 16


def _leaky(h):
    return jnp.where(h >= 0, h, 0.2 * h)


def _enc_body(x_ref, w1_ref, s1_ref, t1_ref, w2_ref, s2_ref, t2_ref,
              w3_ref, b3_ref, z_ref):
    h = jnp.dot(x_ref[...], w1_ref[...], preferred_element_type=jnp.float32,
                precision=_PREC)
    h = _leaky(h * s1_ref[...] + t1_ref[...])
    h = jnp.dot(h, w2_ref[...], preferred_element_type=jnp.float32,
                precision=_PREC)
    h = _leaky(h * s2_ref[...] + t2_ref[...])
    z = jnp.dot(h, w3_ref[...], preferred_element_type=jnp.float32,
                precision=_PREC) + b3_ref[...]
    z_ref[...] = z


def _vq_body(z_ref, et_ref, idx_ref):
    et = et_ref[...]
    e_sq = jnp.sum(et * et, axis=0, keepdims=True)
    scores = e_sq - 2.0 * jnp.dot(z_ref[...], et,
                                  preferred_element_type=jnp.float32,
                                  precision=_PREC)
    m = jnp.min(scores, axis=1, keepdims=True)
    iota = lax.broadcasted_iota(jnp.int32, scores.shape, 1)
    idx = jnp.min(jnp.where(scores == m, iota, K), axis=1)
    idx_ref[...] = idx.reshape(1, 1, BLK)


def _dec_body(q_ref, z_ref, v1_ref, s1_ref, t1_ref, v2_ref, s2_ref, t2_ref,
              v3_ref, b3_ref, recon_ref, lp_ref):
    q = q_ref[...]
    d = q - z_ref[...]
    lp_ref[...] = jnp.broadcast_to(jnp.sum(d * d), (1, 1, 128))
    h = jnp.dot(q, v1_ref[...], preferred_element_type=jnp.float32,
                precision=_PREC)
    h = _leaky(h * s1_ref[...] + t1_ref[...])
    h = jnp.dot(h, v2_ref[...], preferred_element_type=jnp.float32,
                precision=_PREC)
    h = _leaky(h * s2_ref[...] + t2_ref[...])
    recon_ref[...] = jnp.dot(h, v3_ref[...], preferred_element_type=jnp.float32,
                             precision=_PREC) + b3_ref[...]


def _fin_body(lp_ref, c_ref, loss_ref, perp_ref):
    loss = jnp.sum(lp_ref[...]) * ((1.0 + CC) / (128.0 * B * LATENT))
    loss_ref[...] = loss.reshape(1, 1)
    c = c_ref[0] + c_ref[1]
    lane = lax.broadcasted_iota(jnp.int32, c.shape, 1)
    p = c * (1.0 / B)
    ent = jnp.where(lane == 0, p * jnp.log(p + 1e-10), 0.0)
    perp_ref[...] = jnp.exp(-jnp.sum(ent)).reshape(1, 1)


def _sc_body(table_hbm, idx_hbm, q_hbm, counts_hbm,
             idx_a, idx_b, rows_a, rows_b, ones_v, zeros_v, counts_sh,
             sem_a, sem_b):
    cid = lax.axis_index("c")
    sid = lax.axis_index("s")
    wid = sid * _NC + cid
    base = wid * _BPW

    pltpu.sync_copy(idx_hbm.at[pl.ds(base, _CH)], idx_a)
    pltpu.sync_copy(idx_hbm.at[pl.ds(base + _CH, _CH)], idx_b)
    cp_a = pltpu.async_copy(table_hbm.at[idx_a], rows_a, sem_a)
    cp_b = pltpu.async_copy(table_hbm.at[idx_b], rows_b, sem_b)

    @pl.loop(0, K // _NS)
    def _(r):
        zeros_v[r, :] = jnp.zeros((_CW,), jnp.float32)

    @pl.loop(0, _CH)
    def _(r):
        ones_v[r, :] = jnp.ones((_CW,), jnp.float32)

    pltpu.sync_copy(zeros_v, counts_sh.at[pl.ds(sid * (K // _NS), K // _NS)])
    plsc.subcore_barrier()

    pltpu.sync_copy(ones_v, counts_sh.at[idx_a], add=True)
    pltpu.sync_copy(ones_v, counts_sh.at[idx_b], add=True)
    plsc.subcore_barrier()

    pltpu.sync_copy(counts_sh.at[pl.ds(sid * (K // _NS), K // _NS)],
                    counts_hbm.at[cid, pl.ds(sid * (K // _NS), K // _NS)])

    cp_a.wait()
    cp_b.wait()
    pltpu.sync_copy(rows_a, q_hbm.at[pl.ds(base, _CH)])
    pltpu.sync_copy(rows_b, q_hbm.at[pl.ds(base + _CH, _CH)])


def _enc(x, w1, s1, t1, w2, s2, t2, w3, b3):
    full = lambda shape: pl.BlockSpec(shape, lambda i: tuple(0 for _ in shape))
    return pl.pallas_call(
        _enc_body,
        grid=(NBLK,),
        in_specs=[
            pl.BlockSpec((BLK, D_IN), lambda i: (i, 0)),
            full((D_IN, H1)), full((1, H1)), full((1, H1)),
            full((H1, H2)), full((1, H2)), full((1, H2)),
            full((H2, LATENT)), full((1, LATENT)),
        ],
        out_specs=pl.BlockSpec((BLK, LATENT), lambda i: (i, 0)),
        out_shape=jax.ShapeDtypeStruct((B, LATENT), jnp.float32),
    )(x, w1, s1, t1, w2, s2, t2, w3, b3)


def _vq(z, et):
    return pl.pallas_call(
        _vq_body,
        grid=(NBLK,),
        in_specs=[
            pl.BlockSpec((BLK, LATENT), lambda i: (i, 0)),
            pl.BlockSpec((LATENT, K), lambda i: (0, 0)),
        ],
        out_specs=pl.BlockSpec((1, 1, BLK), lambda i: (i, 0, 0)),
        out_shape=jax.ShapeDtypeStruct((NBLK, 1, BLK), jnp.int32),
    )(z, et)


def _sc_gather_counts(table, idx):
    k = pl.kernel(
        _sc_body,
        mesh=plsc.VectorSubcoreMesh(core_axis_name="c", subcore_axis_name="s"),
        out_type=(jax.ShapeDtypeStruct((B, LATENT), jnp.float32),
                  jax.ShapeDtypeStruct((_NC, K, _CW), jnp.float32)),
        scratch_types=[
            pltpu.VMEM((_CH,), jnp.int32),
            pltpu.VMEM((_CH,), jnp.int32),
            pltpu.VMEM((_CH, LATENT), jnp.float32),
            pltpu.VMEM((_CH, LATENT), jnp.float32),
            pltpu.VMEM((_CH, _CW), jnp.float32),
            pltpu.VMEM((K // _NS, _CW), jnp.float32),
            pltpu.VMEM_SHARED((K, _CW), jnp.float32),
            pltpu.SemaphoreType.DMA,
            pltpu.SemaphoreType.DMA,
        ],
        compiler_params=pltpu.CompilerParams(use_tc_tiling_on_sc=False),
    )
    return k(table, idx)


def _dec(q, z, v1, s1, t1, v2, s2, t2, v3, b3):
    full = lambda shape: pl.BlockSpec(shape, lambda i: tuple(0 for _ in shape))
    return pl.pallas_call(
        _dec_body,
        grid=(NBLK,),
        in_specs=[
            pl.BlockSpec((BLK, LATENT), lambda i: (i, 0)),
            pl.BlockSpec((BLK, LATENT), lambda i: (i, 0)),
            full((LATENT, H2)), full((1, H2)), full((1, H2)),
            full((H2, H1)), full((1, H1)), full((1, H1)),
            full((H1, D_IN)), full((1, D_IN)),
        ],
        out_specs=[
            pl.BlockSpec((BLK, D_IN), lambda i: (i, 0)),
            pl.BlockSpec((1, 1, 128), lambda i: (i, 0, 0)),
        ],
        out_shape=[
            jax.ShapeDtypeStruct((B, D_IN), jnp.float32),
            jax.ShapeDtypeStruct((NBLK, 1, 128), jnp.float32),
        ],
    )(q, z, v1, s1, t1, v2, s2, t2, v3, b3)


def _fin(lp, counts):
    return pl.pallas_call(
        _fin_body,
        out_shape=[
            jax.ShapeDtypeStruct((1, 1), jnp.float32),
            jax.ShapeDtypeStruct((1, 1), jnp.float32),
        ],
    )(lp, counts)


def _affine(layer):
    inv = 1.0 / jnp.sqrt(1.0 + BN_EPS)
    s = layer["gamma"] * inv
    t = layer["b"] * s + layer["beta"]
    return s[None, :], t[None, :]


def kernel(x, params):
    enc = params["enc"]
    dec = params["dec"]
    E = params["codebook"]

    s1, t1 = _affine(enc[0])
    s2, t2 = _affine(enc[1])
    z = _enc(x, enc[0]["W"].T, s1, t1, enc[1]["W"].T, s2, t2,
             enc[2]["W"].T, enc[2]["b"][None, :])

    idx3 = _vq(z, E.T)
    idx = idx3.reshape(B)

    q, counts = _sc_gather_counts(E, idx)

    d1, dt1 = _affine(dec[0])
    d2, dt2 = _affine(dec[1])
    recon, lp = _dec(q, z, dec[0]["W"].T, d1, dt1, dec[1]["W"].T, d2, dt2,
                     dec[2]["W"].T, dec[2]["b"][None, :])

    loss, perp = _fin(lp, counts)
    return recon, loss.reshape(()), perp.reshape(())


# speedup vs baseline: 4.2192x; 1.1017x over previous
import functools

import jax
import jax.numpy as jnp
from jax import lax
from jax.experimental import pallas as pl
from jax.experimental.pallas import tpu as pltpu
from jax.experimental.pallas import tpu_sc as plsc

B = 8192
D_IN = 2048
H1 = 2048
H2 = 1024
LATENT = 256
K = 8192
BLK = 256
NBLK = B // BLK
BN_EPS = 1e-5
CC = 0.25

_PREC = jax.lax.Precision.DEFAULT

_NC = 2
_NS = 16
_NW = _NC * _NS
_BPW = B // _NW
_CH = 128
_CW = 16


def _leaky(h):
    return jnp.where(h >= 0, h, 0.2 * h)


def _mmt(a, w):
    return lax.dot_general(a, w, (((1,), (1,)), ((), ())),
                           preferred_element_type=jnp.float32,
                           precision=_PREC)


def _enc_body(x_ref, w1_ref, s1_ref, t1_ref, w2_ref, s2_ref, t2_ref,
              w3_ref, b3_ref, z_ref):
    h = _leaky(_mmt(x_ref[...], w1_ref[...]) * s1_ref[...] + t1_ref[...])
    h = _leaky(_mmt(h, w2_ref[...]) * s2_ref[...] + t2_ref[...])
    z_ref[...] = _mmt(h, w3_ref[...]) + b3_ref[...]


def _vq_body(z_ref, e_ref, idx_ref, esq_ref):
    @pl.when(pl.program_id(0) == 0)
    def _():
        e = e_ref[...]
        esq_ref[...] = lax.dot_general(
            jnp.ones((1, LATENT), jnp.float32), e * e,
            (((1,), (1,)), ((), ())),
            preferred_element_type=jnp.float32, precision=_PREC)

    scores = esq_ref[...] - 2.0 * _mmt(z_ref[...], e_ref[...])
    m = jnp.min(scores, axis=1, keepdims=True)
    iota = lax.broadcasted_iota(jnp.int32, scores.shape, 1)
    idx = jnp.min(jnp.where(scores == m, iota, K), axis=1)
    idx_ref[...] = idx.reshape(1, 1, BLK)


def _dec_body(q_ref, z_ref, v1_ref, s1_ref, t1_ref, v2_ref, s2_ref, t2_ref,
              v3_ref, b3_ref, recon_ref, lp_ref):
    q = q_ref[...]
    d = q - z_ref[...]
    lp_ref[...] = jnp.broadcast_to(jnp.sum(d * d), (1, 1, 128))
    h = _leaky(_mmt(q, v1_ref[...]) * s1_ref[...] + t1_ref[...])
    h = _leaky(_mmt(h, v2_ref[...]) * s2_ref[...] + t2_ref[...])
    recon_ref[...] = _mmt(h, v3_ref[...]) + b3_ref[...]


def _fin_body(lp_ref, c_ref, loss_ref, perp_ref):
    loss = jnp.sum(lp_ref[...]) * ((1.0 + CC) / (128.0 * B * LATENT))
    loss_ref[...] = loss.reshape(1, 1)
    c = c_ref[0] + c_ref[1]
    lane = lax.broadcasted_iota(jnp.int32, c.shape, 1)
    p = c * (1.0 / B)
    ent = jnp.where(lane == 0, p * jnp.log(p + 1e-10), 0.0)
    perp_ref[...] = jnp.exp(-jnp.sum(ent)).reshape(1, 1)


def _sc_body(table_hbm, idx_hbm, q_hbm, counts_hbm,
             idx_a, idx_b, rows_a, rows_b, ones_v, zeros_v, counts_sh,
             sem_a, sem_b):
    cid = lax.axis_index("c")
    sid = lax.axis_index("s")
    wid = sid * _NC + cid
    base = wid * _BPW

    pltpu.sync_copy(idx_hbm.at[pl.ds(base, _CH)], idx_a)
    pltpu.sync_copy(idx_hbm.at[pl.ds(base + _CH, _CH)], idx_b)
    cp_a = pltpu.async_copy(table_hbm.at[idx_a], rows_a, sem_a)
    cp_b = pltpu.async_copy(table_hbm.at[idx_b], rows_b, sem_b)

    @pl.loop(0, K // _NS)
    def _(r):
        zeros_v[r, :] = jnp.zeros((_CW,), jnp.float32)

    @pl.loop(0, _CH)
    def _(r):
        ones_v[r, :] = jnp.ones((_CW,), jnp.float32)

    pltpu.sync_copy(zeros_v, counts_sh.at[pl.ds(sid * (K // _NS), K // _NS)])
    plsc.subcore_barrier()

    pltpu.sync_copy(ones_v, counts_sh.at[idx_a], add=True)
    pltpu.sync_copy(ones_v, counts_sh.at[idx_b], add=True)
    plsc.subcore_barrier()

    pltpu.sync_copy(counts_sh.at[pl.ds(sid * (K // _NS), K // _NS)],
                    counts_hbm.at[cid, pl.ds(sid * (K // _NS), K // _NS)])

    cp_a.wait()
    cp_b.wait()
    pltpu.sync_copy(rows_a, q_hbm.at[pl.ds(base, _CH)])
    pltpu.sync_copy(rows_b, q_hbm.at[pl.ds(base + _CH, _CH)])


def _enc(x, w1, s1, t1, w2, s2, t2, w3, b3):
    full = lambda shape: pl.BlockSpec(shape, lambda i: tuple(0 for _ in shape))
    return pl.pallas_call(
        _enc_body,
        grid=(NBLK,),
        in_specs=[
            pl.BlockSpec((BLK, D_IN), lambda i: (i, 0)),
            full((H1, D_IN)), full((1, H1)), full((1, H1)),
            full((H2, H1)), full((1, H2)), full((1, H2)),
            full((LATENT, H2)), full((1, LATENT)),
        ],
        out_specs=pl.BlockSpec((BLK, LATENT), lambda i: (i, 0)),
        out_shape=jax.ShapeDtypeStruct((B, LATENT), jnp.float32),
    )(x, w1, s1, t1, w2, s2, t2, w3, b3)


def _vq(z, e):
    return pl.pallas_call(
        _vq_body,
        grid=(NBLK,),
        in_specs=[
            pl.BlockSpec((BLK, LATENT), lambda i: (i, 0)),
            pl.BlockSpec((K, LATENT), lambda i: (0, 0)),
        ],
        out_specs=pl.BlockSpec((1, 1, BLK), lambda i: (i, 0, 0)),
        out_shape=jax.ShapeDtypeStruct((NBLK, 1, BLK), jnp.int32),
        scratch_shapes=[pltpu.VMEM((1, K), jnp.float32)],
    )(z, e)


def _sc_gather_counts(table, idx):
    k = pl.kernel(
        _sc_body,
        mesh=plsc.VectorSubcoreMesh(core_axis_name="c", subcore_axis_name="s"),
        out_type=(jax.ShapeDtypeStruct((B, LATENT), jnp.float32),
                  jax.ShapeDtypeStruct((_NC, K, _CW), jnp.float32)),
        scratch_types=[
            pltpu.VMEM((_CH,), jnp.int32),
            pltpu.VMEM((_CH,), jnp.int32),
            pltpu.VMEM((_CH, LATENT), jnp.float32),
            pltpu.VMEM((_CH, LATENT), jnp.float32),
            pltpu.VMEM((_CH, _CW), jnp.float32),
            pltpu.VMEM((K // _NS, _CW), jnp.float32),
            pltpu.VMEM_SHARED((K, _CW), jnp.float32),
            pltpu.SemaphoreType.DMA,
            pltpu.SemaphoreType.DMA,
        ],
        compiler_params=pltpu.CompilerParams(use_tc_tiling_on_sc=False),
    )
    return k(table, idx)


def _dec(q, z, v1, s1, t1, v2, s2, t2, v3, b3):
    full = lambda shape: pl.BlockSpec(shape, lambda i: tuple(0 for _ in shape))
    return pl.pallas_call(
        _dec_body,
        grid=(NBLK,),
        in_specs=[
            pl.BlockSpec((BLK, LATENT), lambda i: (i, 0)),
            pl.BlockSpec((BLK, LATENT), lambda i: (i, 0)),
            full((H2, LATENT)), full((1, H2)), full((1, H2)),
            full((H1, H2)), full((1, H1)), full((1, H1)),
            full((D_IN, H1)), full((1, D_IN)),
        ],
        out_specs=[
            pl.BlockSpec((BLK, D_IN), lambda i: (i, 0)),
            pl.BlockSpec((1, 1, 128), lambda i: (i, 0, 0)),
        ],
        out_shape=[
            jax.ShapeDtypeStruct((B, D_IN), jnp.float32),
            jax.ShapeDtypeStruct((NBLK, 1, 128), jnp.float32),
        ],
    )(q, z, v1, s1, t1, v2, s2, t2, v3, b3)


def _fin(lp, counts):
    return pl.pallas_call(
        _fin_body,
        out_shape=[
            jax.ShapeDtypeStruct((1, 1), jnp.float32),
            jax.ShapeDtypeStruct((1, 1), jnp.float32),
        ],
    )(lp, counts)


def _affine(layer):
    inv = 1.0 / jnp.sqrt(1.0 + BN_EPS)
    s = layer["gamma"] * inv
    t = layer["b"] * s + layer["beta"]
    return s[None, :], t[None, :]


def kernel(x, params):
    enc = params["enc"]
    dec = params["dec"]
    E = params["codebook"]

    s1, t1 = _affine(enc[0])
    s2, t2 = _affine(enc[1])
    z = _enc(x, enc[0]["W"], s1, t1, enc[1]["W"], s2, t2,
             enc[2]["W"], enc[2]["b"][None, :])

    idx3 = _vq(z, E)
    idx = idx3.reshape(B)

    q, counts = _sc_gather_counts(E, idx)

    d1, dt1 = _affine(dec[0])
    d2, dt2 = _affine(dec[1])
    recon, lp = _dec(q, z, dec[0]["W"], d1, dt1, dec[1]["W"], d2, dt2,
                     dec[2]["W"], dec[2]["b"][None, :])

    loss, perp = _fin(lp, counts)
    return recon, loss.reshape(()), perp.reshape(())
